# baseline (speedup 1.0000x reference)
import functools

import jax
import jax.numpy as jnp
from jax import lax
from jax.experimental import pallas as pl
from jax.experimental.pallas import tpu as pltpu
from jax.experimental.pallas import tpu_sc as plsc

_NG, _P, _NPC, _FDIM = 10, 7, 23, 70
_N = _P * _NPC
_NN = _N * _N
_TOT = _NG * _NN
_CHUNK = 8112
_NV = _CHUNK // 16
_LAST_START = _TOT - 8 - _CHUNK

_ALIGNED_END = (_TOT // 8) * 8
_LAST_START = _ALIGNED_END - _CHUNK
_TAIL_START = _TOT - 16
_NW = 32

_mesh = plsc.VectorSubcoreMesh(core_axis_name="c", subcore_axis_name="s")


@functools.partial(
    pl.kernel,
    mesh=_mesh,
    compiler_params=pltpu.CompilerParams(needs_layout_passes=False),
    out_type=jax.ShapeDtypeStruct((_TOT,), jnp.float32),
    scratch_types=[
        pltpu.VMEM((768,), jnp.float32),
        pltpu.VMEM((256,), jnp.float32),
        pltpu.VMEM((_CHUNK,), jnp.float32),
        pltpu.VMEM((128,), jnp.float32),
        pltpu.VMEM((128,), jnp.float32),
        pltpu.VMEM((16,), jnp.int32),
        pltpu.VMEM((16,), jnp.float32),
        pltpu.SemaphoreType.DMA,
        pltpu.SemaphoreType.DMA,
    ],
)
def _sc_rescale(s1_hbm, w_hbm, wout_hbm, s1_v, rp_v, w_v, tmp_v, tmp2_v,
                tidx_v, tval_v, sem1, sem2):
    nc = 2
    wid = lax.axis_index("s") * nc + lax.axis_index("c")
    is_last = wid == _NW - 1
    start = jnp.where(is_last, _LAST_START, wid * _CHUNK)

    lanes = lax.broadcasted_iota(jnp.int32, (16,), 0)
    minl = jnp.minimum(lanes, 6)

    g0 = start // _NN
    ga = jnp.minimum(g0, _NG - 2)
    pltpu.sync_copy(s1_hbm.at[pl.ds(ga * 384, 768)], s1_v)

    for m in range(2):
        base = m * 384
        s = s1_v[pl.ds(base, 16)]
        for r in range(1, 7):
            s = s + s1_v[pl.ds(base + r * 16, 16)]
        tmp_v[pl.ds(16, 16)] = s
        dh = jnp.zeros((16,), jnp.float32)
        for k in range(7):
            sk = plsc.load_gather(tmp_v, [jnp.full((16,), 16 + k, jnp.int32)])
            dh = dh + sk * s1_v[pl.ds(base + 128 + k * 16, 16)]
        dh = 0.1 * dh
        rdiag = plsc.load_gather(s1_v, [base + 256 + minl * 17])
        t = dh / rdiag
        tmp2_v[pl.ds(16, 16)] = t
        for a in range(7):
            rc_row = s1_v[pl.ds(base + 128 + a * 16, 16)]
            rc_col = plsc.load_gather(s1_v, [base + 128 + minl * 16 + a])
            ta = plsc.load_gather(tmp2_v, [jnp.full((16,), 16 + a, jnp.int32)])
            row = 1.0 + t + (rc_col / rc_row) * ta
            row = jnp.where(lanes == a, 1.0, row)
            rp_v[pl.ds(m * 128 + a * 16, 16)] = row

    pltpu.sync_copy(w_hbm.at[pl.ds(start, _CHUNK)], w_v)

    def body(v, _):
        off = v * 16
        f = jnp.minimum(start + off + lanes, _TOT - 1)
        g = f // _NN
        r = f - g * _NN
        i = r // _N
        j = r - i * _N
        idx = jnp.clip(g - ga, 0, 1) * 128 + (i // _NPC) * 16 + (j // _NPC)
        sc = plsc.load_gather(rp_v, [idx])
        w_v[pl.ds(off, 16)] = w_v[pl.ds(off, 16)] * sc
        return _

    lax.fori_loop(0, _NV, body, None)
    pltpu.sync_copy(w_v, wout_hbm.at[pl.ds(start, _CHUNK)])

    @pl.when(is_last)
    def _():
        tidx_v[...] = _TAIL_START + lanes
        pltpu.async_copy(w_hbm.at[tidx_v], tval_v, sem1).wait()
        f = _TAIL_START + lanes
        g = f // _NN
        r = f - g * _NN
        i = r // _N
        j = r - i * _N
        idx = jnp.clip(g - ga, 0, 1) * 128 + (i // _NPC) * 16 + (j // _NPC)
        sc = plsc.load_gather(rp_v, [idx])
        tval_v[...] = tval_v[...] * sc
        pltpu.async_copy(tval_v, wout_hbm.at[tidx_v], sem2).wait()


def _stage2_kernel(wp_ref, rn_ref, hp_ref, d_ref, theta_ref, hpk_ref):
    d_diag = jnp.sum(d_ref[0] * jnp.eye(_NPC, dtype=jnp.float32), axis=-1)
    r = lax.rsqrt(d_diag)
    a = wp_ref[0] * rn_ref[0] * r[:, :, None] * r[:, None, :]
    ahp = lax.dot_general(a, hp_ref[0],
                          (((2,), (1,)), ((0,), (0,))),
                          preferred_element_type=jnp.float32)
    hpk = lax.dot_general(ahp, theta_ref[...],
                          (((2,), (0,)), ((), ())),
                          preferred_element_type=jnp.float32)
    hpk_ref[0] = 0.1 * hpk


def _stage2(Wp, Rn, Hp, D, theta):
    return pl.pallas_call(
        _stage2_kernel,
        grid=(_NG,),
        in_specs=[
            pl.BlockSpec((1, _P, _NPC, _NPC), lambda i: (i, 0, 0, 0)),
            pl.BlockSpec((1, _P, _NPC, _NPC), lambda i: (i, 0, 0, 0)),
            pl.BlockSpec((1, _P, _NPC, _FDIM), lambda i: (i, 0, 0, 0)),
            pl.BlockSpec((1, _P, _NPC, _NPC), lambda i: (i, 0, 0, 0)),
            pl.BlockSpec((_FDIM, _FDIM), lambda i: (0, 0)),
        ],
        out_specs=pl.BlockSpec((1, _P, _NPC, _FDIM), lambda i: (i, 0, 0, 0)),
        out_shape=jax.ShapeDtypeStruct((_NG, _P, _NPC, _FDIM), jnp.float32),
    )(Wp, Rn, Hp, D, theta)


def kernel(Hc, Rc, Rcs, Wp, Rn, Hp, D, W, theta):
    hc_p = jnp.pad(Hc, ((0, 0), (0, 1), (0, 9)))
    rc_p = jnp.pad(Rc, ((0, 0), (0, 1), (0, 9)), constant_values=1.0)
    rcs_p = jnp.pad(Rcs, ((0, 0), (0, 1), (0, 9)), constant_values=1.0)
    s1 = jnp.stack([hc_p, rc_p, rcs_p], axis=1).reshape(-1)

    w_out_flat = _sc_rescale(s1, W.reshape(-1))
    hp_k = _stage2(Wp, Rn, Hp, D, theta)
    return (w_out_flat.reshape(_NG, _N, _N), hp_k)


# speedup vs baseline: 3.0453x; 3.0453x over previous
import functools

import jax
import jax.numpy as jnp
from jax import lax
from jax.experimental import pallas as pl
from jax.experimental.pallas import tpu as pltpu
from jax.experimental.pallas import tpu_sc as plsc

_NG, _P, _NPC, _FDIM = 10, 7, 23, 70
_N = _P * _NPC
_NN = _N * _N
_TOT = _NG * _NN
_NROWS = _NG * _N
_GROUPS = 201
_GW = 8 * _N
_GBUF = _GW + 24
_GPT = 7
_LEFT_ROW = 8 * _GROUPS
_LEFT_OFF = _LEFT_ROW * _N
_TAIL_START = _TOT - 16
_NW = 32
_ROWPAT = [tuple((16 * v + l) // _NPC for l in range(16)) for v in range(10)]

_mesh = plsc.VectorSubcoreMesh(core_axis_name="c", subcore_axis_name="s")


@functools.partial(
    pl.kernel,
    mesh=_mesh,
    compiler_params=pltpu.CompilerParams(needs_layout_passes=False),
    out_type=jax.ShapeDtypeStruct((_TOT,), jnp.float32),
    scratch_types=[
        pltpu.VMEM((768,), jnp.float32),
        pltpu.VMEM((384,), jnp.float32),
        pltpu.VMEM((_GPT * _GBUF,), jnp.float32),
        pltpu.VMEM((128,), jnp.float32),
        pltpu.VMEM((128,), jnp.float32),
        pltpu.VMEM((16,), jnp.int32),
        pltpu.VMEM((16,), jnp.float32),
        pltpu.SemaphoreType.DMA,
        pltpu.SemaphoreType.DMA,
    ],
)
def _sc_rescale(s1_hbm, w_hbm, wout_hbm, s1_v, rp_v, wg_v, tmp_v, tmp2_v,
                tidx_v, tval_v, sem_in, sem_out):
    nc = 2
    wid = lax.axis_index("s") * nc + lax.axis_index("c")
    is_last = wid == _NW - 1
    sg = jnp.minimum(7 * wid, 6 * wid + 9)

    lanes = lax.broadcasted_iota(jnp.int32, (16,), 0)
    minl = jnp.minimum(lanes, 6)

    in_copies = []
    for gi in range(_GPT):
        gb = jnp.minimum(sg + gi, _GROUPS - 1)
        in_copies.append(pltpu.async_copy(
            w_hbm.at[pl.ds(gb * _GW, _GW)],
            wg_v.at[pl.ds(gi * _GBUF, _GW)], sem_in))

    ga = jnp.clip((sg * 8) // _N, 0, _NG - 2)
    pltpu.sync_copy(s1_hbm.at[pl.ds(ga * 384, 768)], s1_v)

    for m in range(2):
        base = m * 384
        s = s1_v[pl.ds(base, 16)]
        for r in range(1, 7):
            s = s + s1_v[pl.ds(base + r * 16, 16)]
        tmp_v[pl.ds(16, 16)] = s
        dh = jnp.zeros((16,), jnp.float32)
        for k in range(7):
            sk = plsc.load_gather(tmp_v, [jnp.full((16,), 16 + k, jnp.int32)])
            dh = dh + sk * s1_v[pl.ds(base + 128 + k * 16, 16)]
        dh = 0.1 * dh
        rdiag = plsc.load_gather(s1_v, [base + 256 + minl * 17])
        t = dh / rdiag
        tmp2_v[pl.ds(16, 16)] = t
        for a in range(7):
            rc_row = s1_v[pl.ds(base + 128 + a * 16, 16)]
            rc_col = plsc.load_gather(s1_v, [base + 128 + minl * 16 + a])
            ta = plsc.load_gather(tmp2_v, [jnp.full((16,), 16 + a, jnp.int32)])
            row = 1.0 + t + (rc_col / rc_row) * ta
            row = jnp.where(lanes == a, 1.0, row)
            rp_v[pl.ds(16 + m * 128 + a * 16, 16)] = row

    def do_row(wofs, r):
        g = r // _N
        i = r - g * _N
        rbase = 16 + jnp.clip(g - ga, 0, 1) * 128 + (i // _NPC) * 16
        for v in range(10):
            cv = (16 * v) // _NPC
            cut = _NPC * (cv + 1) - 16 * v
            if cut >= 16:
                idxv = (rbase + cv) + lanes * 0
            else:
                idxv = rbase + jnp.where(lanes < cut, cv, cv + 1)
            w = wg_v[pl.ds(wofs + 16 * v, 16)]
            sc = plsc.load_gather(rp_v, [idxv])
            wg_v[pl.ds(wofs + 16 * v, 16)] = w * sc
        w = wg_v[pl.ds(wofs + 160, 16)]
        sc = plsc.load_gather(rp_v, [(rbase + 6) + lanes * 0])
        plsc.store_scatter(wg_v, [wofs + 160 + lanes], w * sc, mask=lanes < 1)

    for gi in range(_GPT):
        gb = jnp.minimum(sg + gi, _GROUPS - 1)
        in_copies[gi].wait()
        for k in range(8):
            do_row(gi * _GBUF + k * _N, gb * 8 + k)
        pltpu.async_copy(wg_v.at[pl.ds(gi * _GBUF, _GW)],
                         wout_hbm.at[pl.ds(gb * _GW, _GW)], sem_out).wait()

    @pl.when(is_last)
    def _():
        pltpu.sync_copy(w_hbm.at[pl.ds(_LEFT_OFF, 320)],
                        wg_v.at[pl.ds(0, 320)])
        do_row(0, _LEFT_ROW)
        do_row(_N, _LEFT_ROW + 1)
        pltpu.sync_copy(wg_v.at[pl.ds(0, 320)],
                        wout_hbm.at[pl.ds(_LEFT_OFF, 320)])
        tidx_v[...] = _TAIL_START + lanes
        pltpu.async_copy(w_hbm.at[tidx_v], tval_v, sem_in).wait()
        f = _TAIL_START + lanes
        g = f // _NN
        r = f - g * _NN
        i = r // _N
        j = r - i * _N
        idx = 16 + jnp.clip(g - ga, 0, 1) * 128 + (i // _NPC) * 16 + (j // _NPC)
        sc = plsc.load_gather(rp_v, [idx])
        tval_v[...] = tval_v[...] * sc
        pltpu.async_copy(tval_v, wout_hbm.at[tidx_v], sem_out).wait()


def _stage2_kernel(wp_ref, rn_ref, hp_ref, d_ref, theta_ref, hpk_ref):
    d_diag = jnp.sum(d_ref[0] * jnp.eye(_NPC, dtype=jnp.float32), axis=-1)
    r = lax.rsqrt(d_diag)
    a = wp_ref[0] * rn_ref[0] * r[:, :, None] * r[:, None, :]
    ahp = lax.dot_general(a, hp_ref[0],
                          (((2,), (1,)), ((0,), (0,))),
                          preferred_element_type=jnp.float32)
    hpk = lax.dot_general(ahp, theta_ref[...],
                          (((2,), (0,)), ((), ())),
                          preferred_element_type=jnp.float32)
    hpk_ref[0] = 0.1 * hpk


def _stage2(Wp, Rn, Hp, D, theta):
    return pl.pallas_call(
        _stage2_kernel,
        grid=(_NG,),
        in_specs=[
            pl.BlockSpec((1, _P, _NPC, _NPC), lambda i: (i, 0, 0, 0)),
            pl.BlockSpec((1, _P, _NPC, _NPC), lambda i: (i, 0, 0, 0)),
            pl.BlockSpec((1, _P, _NPC, _FDIM), lambda i: (i, 0, 0, 0)),
            pl.BlockSpec((1, _P, _NPC, _NPC), lambda i: (i, 0, 0, 0)),
            pl.BlockSpec((_FDIM, _FDIM), lambda i: (0, 0)),
        ],
        out_specs=pl.BlockSpec((1, _P, _NPC, _FDIM), lambda i: (i, 0, 0, 0)),
        out_shape=jax.ShapeDtypeStruct((_NG, _P, _NPC, _FDIM), jnp.float32),
    )(Wp, Rn, Hp, D, theta)


def kernel(Hc, Rc, Rcs, Wp, Rn, Hp, D, W, theta):
    hc_p = jnp.pad(Hc, ((0, 0), (0, 1), (0, 9)))
    rc_p = jnp.pad(Rc, ((0, 0), (0, 1), (0, 9)), constant_values=1.0)
    rcs_p = jnp.pad(Rcs, ((0, 0), (0, 1), (0, 9)), constant_values=1.0)
    s1 = jnp.stack([hc_p, rc_p, rcs_p], axis=1).reshape(-1)

    w_out_flat = _sc_rescale(s1, W.reshape(-1))
    hp_k = _stage2(Wp, Rn, Hp, D, theta)
    return (w_out_flat.reshape(_NG, _N, _N), hp_k)
